# baseline (speedup 1.0000x reference)
import jax
import jax.numpy as jnp
from jax import lax
from jax.experimental import pallas as pl
from jax.experimental.pallas import tpu as pltpu


def _mix_kernel(filt_ref, a0_ref, a1_ref, a2_ref, a3_ref, out_ref):
    a = (a0_ref, a1_ref, a2_ref, a3_ref)
    F = out_ref.shape[0]
    tm, N = out_ref.shape[1], out_ref.shape[2]
    t = pl.program_id(0)
    rows = lax.broadcasted_iota(jnp.int32, (tm, N), 0) + t * tm
    cols = lax.broadcasted_iota(jnp.int32, (tm, N), 1)
    diag = (rows == cols).astype(jnp.float32)
    for f in range(F):
        acc = filt_ref[f, 0] * a[0][0]
        for e in range(1, 4):
            acc = acc + filt_ref[f, e] * a[e][0]
        acc = acc + filt_ref[f, 4] * diag
        out_ref[f] = acc.astype(jnp.bfloat16)


def _mix(filt, A, tm=256):
    _, N, _ = A.shape
    tm = min(tm, N)
    F = filt.shape[0]

    def chan(e):
        return pl.BlockSpec((1, tm, N), lambda t, e=e: (e, t, 0))

    return pl.pallas_call(
        _mix_kernel,
        out_shape=jax.ShapeDtypeStruct((F, N, N), jnp.bfloat16),
        grid=(N // tm,),
        in_specs=[
            pl.BlockSpec(memory_space=pltpu.MemorySpace.SMEM),
            chan(0), chan(1), chan(2), chan(3),
        ],
        out_specs=pl.BlockSpec((F, tm, N), lambda t: (0, t, 0)),
        compiler_params=pltpu.CompilerParams(dimension_semantics=("parallel",)),
    )(filt, A, A, A, A)


def _mega_kernel(a_ref, b_ref, feat_ref, gw_ref, gb_ref, w1_ref, o_ref,
                 h0, h1, ideg, odeg, featb):
    p = pl.program_id(1)
    j = pl.program_id(2)
    tn = b_ref.shape[2]

    def colnorm(m):
        deg = jnp.sum(m, axis=0, keepdims=True)
        inv = jnp.where(deg == 0.0, 0.0, 1.0 / deg)
        return (m * inv).astype(jnp.bfloat16)

    @pl.when(p == 0)
    def _layer0():
        m = jnp.dot(a_ref[0], b_ref[0], preferred_element_type=jnp.float32)
        h0[:, pl.ds(j * tn, tn)] = colnorm(m)

    @pl.when(p == 1)
    def _layer1():
        m = jnp.dot(h0[...], b_ref[0], preferred_element_type=jnp.float32)
        h1[:, pl.ds(j * tn, tn)] = colnorm(m)
        nz = (m != 0.0).astype(jnp.float32)
        ideg[:, pl.ds(j * tn, tn)] = jnp.sum(nz, axis=0, keepdims=True)
        rc = jnp.sum(nz, axis=1, keepdims=True)

        @pl.when(j == 0)
        def _():
            odeg[...] = rc

        @pl.when(j > 0)
        def _():
            odeg[...] += rc

    @pl.when(p == 2)
    def _gcn():
        @pl.when(j == 0)
        def _():
            od = jnp.transpose(jnp.maximum(odeg[...], 1.0))
            featb[...] = (feat_ref[...] * lax.rsqrt(od)).astype(jnp.bfloat16)

        mblk = h1[:, pl.ds(j * tn, tn)]
        aggT = lax.dot_general(featb[...], mblk,
                               dimension_numbers=(((1,), (0,)), ((), ())),
                               preferred_element_type=jnp.float32)
        idg = jnp.maximum(ideg[:, pl.ds(j * tn, tn)], 1.0)
        aggT = aggT * lax.rsqrt(idg)
        xT = jnp.maximum(
            lax.dot_general(gw_ref[...], aggT.astype(jnp.bfloat16),
                            dimension_numbers=(((0,), (0,)), ((), ())),
                            preferred_element_type=jnp.float32) + gb_ref[...],
            0.0)
        o_ref[0] = lax.dot_general(w1_ref[...], xT.astype(jnp.bfloat16),
                                   dimension_numbers=(((0,), (0,)), ((), ())),
                                   preferred_element_type=jnp.float32)


def _mega(M6, C, featT, gcn_w, gcn_bT, lin1_w, tn=1024):
    N = M6.shape[1]
    tn = min(tn, N)
    J = N // tn
    w_in = featT.shape[0]
    w_out = gcn_w.shape[1]
    return pl.pallas_call(
        _mega_kernel,
        out_shape=jax.ShapeDtypeStruct((C, w_out, N), jnp.float32),
        grid=(C, 3, J),
        in_specs=[
            pl.BlockSpec((1, N, N), lambda c, p, j: (c, 0, 0)),
            pl.BlockSpec((1, N, tn),
                         lambda c, p, j: (C + jnp.minimum(p, 1) * C + c, 0,
                                          jnp.where(p == 2, J - 1, j))),
            pl.BlockSpec((w_in, N), lambda c, p, j: (0, 0)),
            pl.BlockSpec((w_in, w_out), lambda c, p, j: (0, 0)),
            pl.BlockSpec((w_out, 1), lambda c, p, j: (0, 0)),
            pl.BlockSpec((w_out, w_out), lambda c, p, j: (c, 0)),
        ],
        out_specs=pl.BlockSpec(
            (1, w_out, tn),
            lambda c, p, j: (c, 0, jnp.where(p == 2, j, 0))),
        scratch_shapes=[
            pltpu.VMEM((N, N), jnp.bfloat16),
            pltpu.VMEM((N, N), jnp.bfloat16),
            pltpu.VMEM((1, N), jnp.float32),
            pltpu.VMEM((N, 1), jnp.float32),
            pltpu.VMEM((w_in, N), jnp.bfloat16),
        ],
        compiler_params=pltpu.CompilerParams(
            dimension_semantics=("parallel", "arbitrary", "arbitrary")),
    )(M6, M6, featT, gcn_w.astype(jnp.bfloat16), gcn_bT,
      lin1_w.astype(jnp.bfloat16))


def _combine_kernel(p_ref, b1_ref, w2_ref, b2_ref, y_ref):
    h1T = p_ref[0]
    for c in range(1, p_ref.shape[0]):
        h1T = h1T + p_ref[c]
    h1T = jnp.maximum(h1T + b1_ref[...], 0.0)
    y_ref[...] = lax.dot_general(h1T.astype(jnp.bfloat16), w2_ref[...],
                                 dimension_numbers=(((0,), (0,)), ((), ())),
                                 preferred_element_type=jnp.float32) + b2_ref[...]


def _combine(parts, lin1_bT, lin2_w, lin2_b):
    C, w_out, N = parts.shape
    num_class = lin2_w.shape[1]
    return pl.pallas_call(
        _combine_kernel,
        out_shape=jax.ShapeDtypeStruct((N, num_class), jnp.float32),
        in_specs=[
            pl.BlockSpec((C, w_out, N), lambda: (0, 0, 0)),
            pl.BlockSpec((w_out, 1), lambda: (0, 0)),
            pl.BlockSpec((w_out, num_class), lambda: (0, 0)),
            pl.BlockSpec((1, num_class), lambda: (0, 0)),
        ],
        out_specs=pl.BlockSpec((N, num_class), lambda: (0, 0)),
    )(parts, lin1_bT, lin2_w.astype(jnp.bfloat16), lin2_b)


def kernel(A, h, gt0_w1, gt0_w2, gt1_w1,
           gcn_w, gcn_b, lin1_w, lin1_b, lin2_w, lin2_b):
    C = gt0_w1.shape[0]
    f1 = jax.nn.softmax(gt0_w1, axis=1)
    f2 = jax.nn.softmax(gt0_w2, axis=1)
    fl1 = jax.nn.softmax(gt1_w1, axis=1)
    filt = jnp.concatenate([f1, f2, fl1], axis=0)

    M6 = _mix(filt, A)

    parts = _mega(M6, C, h.T, gcn_w, gcn_b.T, lin1_w)
    return _combine(parts, lin1_b.T, lin2_w, lin2_b)


# speedup vs baseline: 1.0051x; 1.0051x over previous
import jax
import jax.numpy as jnp
from jax import lax
from jax.experimental import pallas as pl
from jax.experimental.pallas import tpu as pltpu


def _mix_kernel(filt_ref, a0_ref, a1_ref, a2_ref, a3_ref, out_ref):
    a = (a0_ref, a1_ref, a2_ref, a3_ref)
    F = out_ref.shape[0]
    tm, N = out_ref.shape[1], out_ref.shape[2]
    t = pl.program_id(0)
    rows = lax.broadcasted_iota(jnp.int32, (tm, N), 0) + t * tm
    cols = lax.broadcasted_iota(jnp.int32, (tm, N), 1)
    diag = (rows == cols).astype(jnp.float32)
    for f in range(F):
        acc = filt_ref[f, 0] * a[0][0]
        for e in range(1, 4):
            acc = acc + filt_ref[f, e] * a[e][0]
        acc = acc + filt_ref[f, 4] * diag
        out_ref[f] = acc.astype(jnp.bfloat16)


def _mix(filt, A, tm=256):
    _, N, _ = A.shape
    tm = min(tm, N)
    F = filt.shape[0]

    def chan(e):
        return pl.BlockSpec((1, tm, N), lambda t, e=e: (e, t, 0))

    return pl.pallas_call(
        _mix_kernel,
        out_shape=jax.ShapeDtypeStruct((F, N, N), jnp.bfloat16),
        grid=(N // tm,),
        in_specs=[
            pl.BlockSpec(memory_space=pltpu.MemorySpace.SMEM),
            chan(0), chan(1), chan(2), chan(3),
        ],
        out_specs=pl.BlockSpec((F, tm, N), lambda t: (0, t, 0)),
        compiler_params=pltpu.CompilerParams(dimension_semantics=("parallel",)),
    )(filt, A, A, A, A)


def _mega_kernel(a_ref, b_ref, feat_ref, gw_ref, gb_ref, w1_ref, o_ref,
                 h0, h1, ideg, odeg, featb):
    p = pl.program_id(1)
    j = pl.program_id(2)
    tn = b_ref.shape[2]

    def colnorm(m):
        deg = jnp.sum(m, axis=0, keepdims=True)
        inv = jnp.where(deg == 0.0, 0.0, 1.0 / deg)
        return (m * inv).astype(jnp.bfloat16)

    @pl.when(p == 0)
    def _layer0():
        m = jnp.dot(a_ref[0], b_ref[0], preferred_element_type=jnp.float32)
        h0[:, pl.ds(j * tn, tn)] = colnorm(m)

    @pl.when(p == 1)
    def _layer1():
        m = jnp.dot(h0[...], b_ref[0], preferred_element_type=jnp.float32)
        h1[:, pl.ds(j * tn, tn)] = colnorm(m)
        nz = (m != 0.0).astype(jnp.float32)
        ideg[:, pl.ds(j * tn, tn)] = jnp.sum(nz, axis=0, keepdims=True)
        rc = jnp.sum(nz, axis=1, keepdims=True)

        @pl.when(j == 0)
        def _():
            odeg[...] = rc

        @pl.when(j > 0)
        def _():
            odeg[...] += rc

    @pl.when(p == 2)
    def _gcn():
        @pl.when(j == 0)
        def _():
            od = jnp.maximum(odeg[...], 1.0)
            featb[...] = (feat_ref[...] * lax.rsqrt(od)).astype(jnp.bfloat16)

        mblk = h1[:, pl.ds(j * tn, tn)]
        agg = lax.dot_general(mblk, featb[...],
                              dimension_numbers=(((0,), (0,)), ((), ())),
                              preferred_element_type=jnp.float32)
        idg = jnp.maximum(ideg[:, pl.ds(j * tn, tn)], 1.0)
        agg = agg * lax.rsqrt(idg).T
        x = jnp.maximum(
            jnp.dot(agg.astype(jnp.bfloat16), gw_ref[...],
                    preferred_element_type=jnp.float32) + gb_ref[...], 0.0)
        o_ref[0] = jnp.dot(x.astype(jnp.bfloat16), w1_ref[...],
                           preferred_element_type=jnp.float32)


def _mega(M6, C, feat, gcn_w, gcn_b, lin1_w, tn=1024):
    N = M6.shape[1]
    tn = min(tn, N)
    J = N // tn
    w_in = feat.shape[1]
    w_out = gcn_w.shape[1]
    return pl.pallas_call(
        _mega_kernel,
        out_shape=jax.ShapeDtypeStruct((C, N, w_out), jnp.float32),
        grid=(C, 3, J),
        in_specs=[
            pl.BlockSpec((1, N, N), lambda c, p, j: (c, 0, 0)),
            pl.BlockSpec((1, N, tn),
                         lambda c, p, j: (C + jnp.minimum(p, 1) * C + c, 0,
                                          jnp.where(p == 2, J - 1, j))),
            pl.BlockSpec((N, w_in), lambda c, p, j: (0, 0)),
            pl.BlockSpec((w_in, w_out), lambda c, p, j: (0, 0)),
            pl.BlockSpec((1, w_out), lambda c, p, j: (0, 0)),
            pl.BlockSpec((w_out, w_out), lambda c, p, j: (c, 0)),
        ],
        out_specs=pl.BlockSpec(
            (1, tn, w_out),
            lambda c, p, j: (c, jnp.where(p == 2, j, 0), 0)),
        scratch_shapes=[
            pltpu.VMEM((N, N), jnp.bfloat16),
            pltpu.VMEM((N, N), jnp.bfloat16),
            pltpu.VMEM((1, N), jnp.float32),
            pltpu.VMEM((N, 1), jnp.float32),
            pltpu.VMEM((N, 128), jnp.bfloat16),
        ],
        compiler_params=pltpu.CompilerParams(
            dimension_semantics=("parallel", "arbitrary", "arbitrary")),
    )(M6, M6, feat, gcn_w.astype(jnp.bfloat16), gcn_b,
      lin1_w.astype(jnp.bfloat16))


def _combine_kernel(p_ref, b1_ref, w2_ref, b2_ref, y_ref):
    h1 = p_ref[0]
    for c in range(1, p_ref.shape[0]):
        h1 = h1 + p_ref[c]
    h1 = jnp.maximum(h1 + b1_ref[...], 0.0)
    y_ref[...] = jnp.dot(h1.astype(jnp.bfloat16), w2_ref[...],
                         preferred_element_type=jnp.float32) + b2_ref[...]


def _combine(parts, lin1_b, lin2_w, lin2_b):
    C, N, w_out = parts.shape
    num_class = lin2_w.shape[1]
    return pl.pallas_call(
        _combine_kernel,
        out_shape=jax.ShapeDtypeStruct((N, num_class), jnp.float32),
        in_specs=[
            pl.BlockSpec((C, N, w_out), lambda: (0, 0, 0)),
            pl.BlockSpec((1, w_out), lambda: (0, 0)),
            pl.BlockSpec((w_out, num_class), lambda: (0, 0)),
            pl.BlockSpec((1, num_class), lambda: (0, 0)),
        ],
        out_specs=pl.BlockSpec((N, num_class), lambda: (0, 0)),
    )(parts, lin1_b, lin2_w.astype(jnp.bfloat16), lin2_b)


def kernel(A, h, gt0_w1, gt0_w2, gt1_w1,
           gcn_w, gcn_b, lin1_w, lin1_b, lin2_w, lin2_b):
    C = gt0_w1.shape[0]
    f1 = jax.nn.softmax(gt0_w1, axis=1)
    f2 = jax.nn.softmax(gt0_w2, axis=1)
    fl1 = jax.nn.softmax(gt1_w1, axis=1)
    filt = jnp.concatenate([f1, f2, fl1], axis=0)

    M6 = _mix(filt, A)

    parts = _mega(M6, C, h, gcn_w, gcn_b, lin1_w)
    return _combine(parts, lin1_b, lin2_w, lin2_b)
